# Initial kernel scaffold; baseline (speedup 1.0000x reference)
import jax
import jax.numpy as jnp
from jax.experimental import pallas as pl


def kernel(x, pos, W_qkv, b_qkv, W_proj, b_proj, W_pe, b_pe):
    raise NotImplementedError("write your pallas kernel here")


# speedup vs baseline: 5.6879x; 5.6879x over previous
import functools
import math

import jax
import jax.numpy as jnp
from jax.experimental import pallas as pl
from jax.experimental.pallas import tpu as pltpu

DIM = 2048
H = 16
M = 16
E = DIM // H
TOPK = 2
PE_PAD = 128


def _bf(a):
    return a.astype(jnp.bfloat16)


def _dot_t(a, b):
    return jax.lax.dot_general(
        _bf(a), _bf(b), (((1,), (1,)), ((), ())),
        preferred_element_type=jnp.float32)


def _qkv_body(x_ref, posp_ref, wpet_ref, bpe_ref, w_ref, b_ref, o_ref, xpe_ref):
    j = pl.program_id(1)

    @pl.when(j == 0)
    def _():
        p = posp_ref[...]
        bn = p.shape[0]
        pm = jnp.mean(p.reshape(bn // M, M, PE_PAD), axis=1, keepdims=True)
        rel = (p.reshape(bn // M, M, PE_PAD) - pm).reshape(bn, PE_PAD)
        pe = jax.lax.dot_general(
            _bf(rel), _bf(wpet_ref[...]), (((1,), (0,)), ((), ())),
            preferred_element_type=jnp.float32)
        xpe_ref[...] = x_ref[...] + pe + bpe_ref[...]

    o_ref[...] = _dot_t(xpe_ref[...], w_ref[...]) + b_ref[...]


def _attn_body(q_ref, k_ref, v_ref, o_ref, *, bn, n_tokens):
    kh = k_ref[...]
    qh = q_ref[...]
    means = jnp.mean(kh.reshape(n_tokens // M, M, E), axis=1)
    sim = _dot_t(qh, means)
    nb = n_tokens // M
    j1 = jnp.argmax(sim, axis=1)
    col = jax.lax.broadcasted_iota(jnp.int32, (bn, nb), 1)
    sim2 = jnp.where(col == j1[:, None], -jnp.inf, sim)
    j2 = jnp.argmax(sim2, axis=1)

    scores = _dot_t(qh, kh) * (1.0 / math.sqrt(E))
    keyball = jax.lax.broadcasted_iota(jnp.int32, (bn, n_tokens), 1) // M
    sel = (keyball == j1[:, None]) | (keyball == j2[:, None])
    masked = jnp.where(sel, scores, -jnp.inf)
    mx = jnp.max(masked, axis=1, keepdims=True)
    e = jnp.exp(masked - mx)
    s = jnp.sum(e, axis=1, keepdims=True)
    p = e / s
    o_ref[...] = jax.lax.dot_general(
        _bf(p), _bf(v_ref[...]), (((1,), (0,)), ((), ())),
        preferred_element_type=jnp.float32)


def _proj_body(a_ref, w_ref, b_ref, o_ref):
    o_ref[...] = _dot_t(a_ref[...], w_ref[...]) + b_ref[...]


@jax.jit
def kernel(x, pos, W_qkv, b_qkv, W_proj, b_proj, W_pe, b_pe):
    n_tokens = x.shape[0]

    Wr = W_qkv.reshape(H, E, 3, DIM)
    Wcat = jnp.transpose(Wr, (2, 0, 1, 3)).reshape(3 * DIM, DIM)
    br = b_qkv.reshape(H, E, 3)
    bcat = jnp.transpose(br, (2, 0, 1)).reshape(1, 3 * DIM)
    posp = jnp.pad(pos, ((0, 0), (0, PE_PAD - pos.shape[1])))
    wpet = jnp.pad(W_pe.T, ((0, PE_PAD - pos.shape[1]), (0, 0)))
    bpe = b_pe.reshape(1, DIM)

    BN_A, BC_A = 512, 512
    qkv = pl.pallas_call(
        _qkv_body,
        grid=(n_tokens // BN_A, 3 * DIM // BC_A),
        in_specs=[
            pl.BlockSpec((BN_A, DIM), lambda i, j: (i, 0)),
            pl.BlockSpec((BN_A, PE_PAD), lambda i, j: (i, 0)),
            pl.BlockSpec((PE_PAD, DIM), lambda i, j: (0, 0)),
            pl.BlockSpec((1, DIM), lambda i, j: (0, 0)),
            pl.BlockSpec((BC_A, DIM), lambda i, j: (j, 0)),
            pl.BlockSpec((1, BC_A), lambda i, j: (0, j)),
        ],
        out_specs=pl.BlockSpec((BN_A, BC_A), lambda i, j: (i, j)),
        out_shape=jax.ShapeDtypeStruct((n_tokens, 3 * DIM), jnp.float32),
        scratch_shapes=[pltpu.VMEM((BN_A, DIM), jnp.float32)],
        compiler_params=pltpu.CompilerParams(
            dimension_semantics=("parallel", "arbitrary")),
    )(x, posp, wpet, bpe, Wcat, bcat)

    q = qkv[:, :DIM]
    k = qkv[:, DIM:2 * DIM]
    v = qkv[:, 2 * DIM:]

    BN_B = 512
    attn_out = pl.pallas_call(
        functools.partial(_attn_body, bn=BN_B, n_tokens=n_tokens),
        grid=(H, n_tokens // BN_B),
        in_specs=[
            pl.BlockSpec((BN_B, E), lambda h, i: (i, h)),
            pl.BlockSpec((n_tokens, E), lambda h, i: (0, h)),
            pl.BlockSpec((n_tokens, E), lambda h, i: (0, h)),
        ],
        out_specs=pl.BlockSpec((BN_B, E), lambda h, i: (i, h)),
        out_shape=jax.ShapeDtypeStruct((n_tokens, DIM), jnp.float32),
        compiler_params=pltpu.CompilerParams(
            dimension_semantics=("parallel", "parallel")),
    )(q, k, v)

    BN_C, BC_C = 512, 1024
    out = pl.pallas_call(
        _proj_body,
        grid=(n_tokens // BN_C, DIM // BC_C),
        in_specs=[
            pl.BlockSpec((BN_C, DIM), lambda i, j: (i, 0)),
            pl.BlockSpec((BC_C, DIM), lambda i, j: (j, 0)),
            pl.BlockSpec((1, BC_C), lambda i, j: (0, j)),
        ],
        out_specs=pl.BlockSpec((BN_C, BC_C), lambda i, j: (i, j)),
        out_shape=jax.ShapeDtypeStruct((n_tokens, DIM), jnp.float32),
        compiler_params=pltpu.CompilerParams(
            dimension_semantics=("parallel", "parallel")),
    )(attn_out, W_proj, b_proj.reshape(1, DIM))
    return out


# speedup vs baseline: 6.0367x; 1.0613x over previous
import functools
import math

import jax
import jax.numpy as jnp
from jax.experimental import pallas as pl
from jax.experimental.pallas import tpu as pltpu

DIM = 2048
H = 16
M = 16
E = DIM // H
TOPK = 2
PE_PAD = 128


def _bf(a):
    return a.astype(jnp.bfloat16)


def _dot_t(a, b):
    return jax.lax.dot_general(
        _bf(a), _bf(b), (((1,), (1,)), ((), ())),
        preferred_element_type=jnp.float32)


def _qkv_body(x_ref, posp_ref, wpet_ref, bpe_ref, w_ref, b_ref, o_ref, xpe_ref):
    j = pl.program_id(1)

    @pl.when(j == 0)
    def _():
        p = posp_ref[...]
        bn = p.shape[0]
        pm = jnp.mean(p.reshape(bn // M, M, PE_PAD), axis=1, keepdims=True)
        rel = (p.reshape(bn // M, M, PE_PAD) - pm).reshape(bn, PE_PAD)
        pe = jax.lax.dot_general(
            _bf(rel), _bf(wpet_ref[...]), (((1,), (0,)), ((), ())),
            preferred_element_type=jnp.float32)
        xpe_ref[...] = x_ref[...] + pe + bpe_ref[...]

    o_ref[...] = _dot_t(xpe_ref[...], w_ref[...]) + b_ref[...]


def _attn_body(q_ref, k_ref, v_ref, o_ref, *, bn, n_tokens):
    kh = k_ref[...]
    qh = q_ref[...]
    means = jnp.mean(kh.reshape(n_tokens // M, M, E), axis=1)
    sim = _dot_t(qh, means)
    nb = n_tokens // M
    j1 = jnp.argmax(sim, axis=1)
    col = jax.lax.broadcasted_iota(jnp.int32, (bn, nb), 1)
    sim2 = jnp.where(col == j1[:, None], -jnp.inf, sim)
    j2 = jnp.argmax(sim2, axis=1)

    scores = _dot_t(qh, kh) * (1.0 / math.sqrt(E))
    keyball = jax.lax.broadcasted_iota(jnp.int32, (bn, n_tokens), 1) // M
    sel = (keyball == j1[:, None]) | (keyball == j2[:, None])
    masked = jnp.where(sel, scores, -jnp.inf)
    mx = jnp.max(masked, axis=1, keepdims=True)
    e = jnp.exp(masked - mx)
    s = jnp.sum(e, axis=1, keepdims=True)
    p = e / s
    o_ref[...] = jax.lax.dot_general(
        _bf(p), _bf(v_ref[...]), (((1,), (0,)), ((), ())),
        preferred_element_type=jnp.float32)


def _proj_body(a_ref, w_ref, b_ref, o_ref):
    o_ref[...] = _dot_t(a_ref[...], w_ref[...]) + b_ref[...]


@jax.jit
def kernel(x, pos, W_qkv, b_qkv, W_proj, b_proj, W_pe, b_pe):
    n_tokens = x.shape[0]

    Wr = W_qkv.reshape(H, E, 3, DIM)
    Wcat = jnp.transpose(Wr, (2, 0, 1, 3)).reshape(3 * DIM, DIM)
    br = b_qkv.reshape(H, E, 3)
    bcat = jnp.transpose(br, (2, 0, 1)).reshape(1, 3 * DIM)
    posp = jnp.pad(pos, ((0, 0), (0, PE_PAD - pos.shape[1])))
    wpet = jnp.pad(W_pe.T, ((0, PE_PAD - pos.shape[1]), (0, 0)))
    bpe = b_pe.reshape(1, DIM)

    BN_A, BC_A = 512, 512
    qkv = pl.pallas_call(
        _qkv_body,
        grid=(n_tokens // BN_A, 3 * DIM // BC_A),
        in_specs=[
            pl.BlockSpec((BN_A, DIM), lambda i, j: (i, 0)),
            pl.BlockSpec((BN_A, PE_PAD), lambda i, j: (i, 0)),
            pl.BlockSpec((PE_PAD, DIM), lambda i, j: (0, 0)),
            pl.BlockSpec((1, DIM), lambda i, j: (0, 0)),
            pl.BlockSpec((BC_A, DIM), lambda i, j: (j, 0)),
            pl.BlockSpec((1, BC_A), lambda i, j: (0, j)),
        ],
        out_specs=pl.BlockSpec((BN_A, BC_A), lambda i, j: (i, j)),
        out_shape=jax.ShapeDtypeStruct((n_tokens, 3 * DIM), jnp.float32),
        scratch_shapes=[pltpu.VMEM((BN_A, DIM), jnp.float32)],
        compiler_params=pltpu.CompilerParams(
            dimension_semantics=("parallel", "arbitrary")),
    )(x, posp, wpet, bpe, Wcat, bcat)

    BN_B = 512
    attn_out = pl.pallas_call(
        functools.partial(_attn_body, bn=BN_B, n_tokens=n_tokens),
        grid=(H, n_tokens // BN_B),
        in_specs=[
            pl.BlockSpec((BN_B, E), lambda h, i: (i, h)),
            pl.BlockSpec((n_tokens, E), lambda h, i: (0, H + h)),
            pl.BlockSpec((n_tokens, E), lambda h, i: (0, 2 * H + h)),
        ],
        out_specs=pl.BlockSpec((BN_B, E), lambda h, i: (i, h)),
        out_shape=jax.ShapeDtypeStruct((n_tokens, DIM), jnp.float32),
        compiler_params=pltpu.CompilerParams(
            dimension_semantics=("parallel", "parallel")),
    )(qkv, qkv, qkv)

    BN_C, BC_C = 512, 1024
    out = pl.pallas_call(
        _proj_body,
        grid=(n_tokens // BN_C, DIM // BC_C),
        in_specs=[
            pl.BlockSpec((BN_C, DIM), lambda i, j: (i, 0)),
            pl.BlockSpec((BC_C, DIM), lambda i, j: (j, 0)),
            pl.BlockSpec((1, BC_C), lambda i, j: (0, j)),
        ],
        out_specs=pl.BlockSpec((BN_C, BC_C), lambda i, j: (i, j)),
        out_shape=jax.ShapeDtypeStruct((n_tokens, DIM), jnp.float32),
        compiler_params=pltpu.CompilerParams(
            dimension_semantics=("parallel", "parallel")),
    )(attn_out, W_proj, b_proj.reshape(1, DIM))
    return out


# speedup vs baseline: 6.0690x; 1.0053x over previous
import functools
import math

import jax
import jax.numpy as jnp
from jax.experimental import pallas as pl
from jax.experimental.pallas import tpu as pltpu

DIM = 2048
H = 16
M = 16
E = DIM // H
TOPK = 2
PE_PAD = 128


def _bf(a):
    return a.astype(jnp.bfloat16)


def _dot_nt(a, b):
    return jax.lax.dot_general(
        a, b, (((1,), (1,)), ((), ())), preferred_element_type=jnp.float32)


def _dot_nn(a, b):
    return jax.lax.dot_general(
        a, b, (((1,), (0,)), ((), ())), preferred_element_type=jnp.float32)


def _qkv_body(x_ref, posp_ref, wpet_ref, bpe_ref, w_ref, b_ref,
              qkv_ref, km_ref, xpe_ref, *, n_jk):
    j = pl.program_id(1)

    @pl.when(j == 0)
    def _():
        p = posp_ref[...]
        bn = p.shape[0]
        pm = jnp.mean(p.reshape(bn // M, M, PE_PAD), axis=1, keepdims=True)
        rel = (p.reshape(bn // M, M, PE_PAD) - pm).reshape(bn, PE_PAD)
        pe = _dot_nn(_bf(rel), _bf(wpet_ref[...]))
        xpe_ref[...] = _bf(x_ref[...] + pe + bpe_ref[...])

    of = _dot_nt(xpe_ref[...], w_ref[...]) + b_ref[...]
    qkv_ref[...] = _bf(of)
    bn, bc = of.shape

    @pl.when((j >= n_jk) & (j < 2 * n_jk))
    def _():
        km_ref[:, pl.ds((j - n_jk) * bc, bc)] = jnp.mean(
            of.reshape(bn // M, M, bc), axis=1)


def _attn_body(q_ref, k_ref, v_ref, km_ref, em_ref, o_ref, *, bn, n_tokens):
    qh = q_ref[...]
    sim = _dot_nt(qh, _bf(km_ref[...]))
    nb = n_tokens // M
    j1 = jnp.argmax(sim, axis=1)
    col = jax.lax.broadcasted_iota(jnp.int32, (bn, nb), 1)
    sim2 = jnp.where(col == j1[:, None], -jnp.inf, sim)
    j2 = jnp.argmax(sim2, axis=1)
    selb = (col == j1[:, None]) | (col == j2[:, None])
    bias_b = jnp.where(selb, 0.0, -1e30).astype(jnp.bfloat16)
    bias_k = _dot_nn(bias_b, em_ref[...])

    scores = _dot_nt(qh, k_ref[...]) * (1.0 / math.sqrt(E)) + bias_k
    mx = jnp.max(scores, axis=1, keepdims=True)
    e = jnp.exp(scores - mx)
    s = jnp.sum(e, axis=1, keepdims=True)
    p = _bf(e / s)
    o_ref[...] = _bf(_dot_nn(p, v_ref[...]))


def _proj_body(a_ref, w_ref, b_ref, o_ref):
    o_ref[...] = _dot_nt(a_ref[...], w_ref[...]) + b_ref[...]


@jax.jit
def kernel(x, pos, W_qkv, b_qkv, W_proj, b_proj, W_pe, b_pe):
    n_tokens = x.shape[0]
    nb = n_tokens // M

    Wcat = _bf(W_qkv).reshape(DIM, 3, DIM).transpose(1, 0, 2).reshape(3 * DIM, DIM)
    bcat = b_qkv.reshape(DIM, 3).transpose(1, 0).reshape(1, 3 * DIM)
    posp = jnp.pad(pos, ((0, 0), (0, PE_PAD - pos.shape[1])))
    wpet = jnp.pad(W_pe.T, ((0, PE_PAD - pos.shape[1]), (0, 0)))
    bpe = b_pe.reshape(1, DIM)
    em = (jax.lax.broadcasted_iota(jnp.int32, (nb, n_tokens), 0)
          == jax.lax.broadcasted_iota(jnp.int32, (nb, n_tokens), 1) // M
          ).astype(jnp.bfloat16)

    BN_A, BC_A = 512, 512
    n_jk = DIM // BC_A
    qkv, kmeans = pl.pallas_call(
        functools.partial(_qkv_body, n_jk=n_jk),
        grid=(n_tokens // BN_A, 3 * DIM // BC_A),
        in_specs=[
            pl.BlockSpec((BN_A, DIM), lambda i, j: (i, 0)),
            pl.BlockSpec((BN_A, PE_PAD), lambda i, j: (i, 0)),
            pl.BlockSpec((PE_PAD, DIM), lambda i, j: (0, 0)),
            pl.BlockSpec((1, DIM), lambda i, j: (0, 0)),
            pl.BlockSpec((BC_A, DIM), lambda i, j: (j, 0)),
            pl.BlockSpec((1, BC_A), lambda i, j: (0, j)),
        ],
        out_specs=[
            pl.BlockSpec((BN_A, BC_A), lambda i, j: (i, j)),
            pl.BlockSpec((BN_A // M, DIM), lambda i, j: (i, 0)),
        ],
        out_shape=[
            jax.ShapeDtypeStruct((n_tokens, 3 * DIM), jnp.bfloat16),
            jax.ShapeDtypeStruct((nb, DIM), jnp.float32),
        ],
        scratch_shapes=[pltpu.VMEM((BN_A, DIM), jnp.bfloat16)],
        compiler_params=pltpu.CompilerParams(
            dimension_semantics=("parallel", "arbitrary")),
    )(x, posp, wpet, bpe, Wcat, bcat)

    BN_B = 512
    attn_out = pl.pallas_call(
        functools.partial(_attn_body, bn=BN_B, n_tokens=n_tokens),
        grid=(H, n_tokens // BN_B),
        in_specs=[
            pl.BlockSpec((BN_B, E), lambda h, i: (i, h)),
            pl.BlockSpec((n_tokens, E), lambda h, i: (0, H + h)),
            pl.BlockSpec((n_tokens, E), lambda h, i: (0, 2 * H + h)),
            pl.BlockSpec((nb, E), lambda h, i: (0, h)),
            pl.BlockSpec((nb, n_tokens), lambda h, i: (0, 0)),
        ],
        out_specs=pl.BlockSpec((BN_B, E), lambda h, i: (i, h)),
        out_shape=jax.ShapeDtypeStruct((n_tokens, DIM), jnp.bfloat16),
        compiler_params=pltpu.CompilerParams(
            dimension_semantics=("parallel", "parallel")),
    )(qkv, qkv, qkv, kmeans, em)

    BN_C, BC_C = 512, 1024
    out = pl.pallas_call(
        _proj_body,
        grid=(n_tokens // BN_C, DIM // BC_C),
        in_specs=[
            pl.BlockSpec((BN_C, DIM), lambda i, j: (i, 0)),
            pl.BlockSpec((BC_C, DIM), lambda i, j: (j, 0)),
            pl.BlockSpec((1, BC_C), lambda i, j: (0, j)),
        ],
        out_specs=pl.BlockSpec((BN_C, BC_C), lambda i, j: (i, j)),
        out_shape=jax.ShapeDtypeStruct((n_tokens, DIM), jnp.float32),
        compiler_params=pltpu.CompilerParams(
            dimension_semantics=("parallel", "parallel")),
    )(attn_out, _bf(W_proj), b_proj.reshape(1, DIM))
    return out
